# baseline (speedup 1.0000x reference)
import jax
import jax.numpy as jnp
from jax import lax
from jax.experimental import pallas as pl

_BC = 200


def _masked_matmul_kernel(xt_ref, table_ref, o_ref):
    c = pl.program_id(0)
    mask = (xt_ref[...] != 0).astype(jnp.float32)
    part = lax.dot_general(
        mask, table_ref[...],
        dimension_numbers=(((0,), (0,)), ((), ())),
        preferred_element_type=jnp.float32,
    )

    @pl.when(c == 0)
    def _init():
        o_ref[...] = part

    @pl.when(c != 0)
    def _acc():
        o_ref[...] += part


@jax.jit
def kernel(x, table):
    batch, num_cat = x.shape
    _, embed_dim = table.shape
    xt = x.T
    return pl.pallas_call(
        _masked_matmul_kernel,
        grid=(num_cat // _BC,),
        in_specs=[
            pl.BlockSpec((_BC, batch), lambda c: (c, 0)),
            pl.BlockSpec((_BC, embed_dim), lambda c: (c, 0)),
        ],
        out_specs=pl.BlockSpec((batch, embed_dim), lambda c: (0, 0)),
        out_shape=jax.ShapeDtypeStruct((batch, embed_dim), jnp.float32),
    )(xt, table)


# speedup vs baseline: 1.1380x; 1.1380x over previous
import jax
import jax.numpy as jnp
from jax import lax
from jax.experimental import pallas as pl

_STEP = 2048
_SUB = 512
_NSUB = _STEP // _SUB


def _masked_matmul_kernel(*refs):
    xt_refs = refs[:_NSUB]
    table_ref = refs[_NSUB]
    o_ref = refs[_NSUB + 1]
    t = table_ref[...]
    for j in range(_NSUB):
        mask = (xt_refs[j][...] != 0).astype(jnp.float32)
        o_ref[j * _SUB:(j + 1) * _SUB, :] = lax.dot_general(
            mask, t,
            dimension_numbers=(((0,), (0,)), ((), ())),
            preferred_element_type=jnp.float32,
        )


@jax.jit
def kernel(x, table):
    batch, num_cat = x.shape
    _, embed_dim = table.shape
    xt = x.T
    in_specs = [
        pl.BlockSpec((num_cat, _SUB), (lambda i, j=j: (0, i * _NSUB + j)))
        for j in range(_NSUB)
    ]
    in_specs.append(pl.BlockSpec((num_cat, embed_dim), lambda i: (0, 0)))
    return pl.pallas_call(
        _masked_matmul_kernel,
        grid=(batch // _STEP,),
        in_specs=in_specs,
        out_specs=pl.BlockSpec((_STEP, embed_dim), lambda i: (i, 0)),
        out_shape=jax.ShapeDtypeStruct((batch, embed_dim), jnp.float32),
    )(*([xt] * _NSUB), table)
